# baseline (speedup 1.0000x reference)
import functools

import jax
import jax.numpy as jnp
from jax import lax
from jax.experimental import pallas as pl
from jax.experimental.pallas import tpu as pltpu
from jax.experimental.pallas import tpu_sc as plsc

SEQ = 200
SEQ_PAD = 256
EMBED = 64
LANES = 16
NUM_WORKERS = 32
CHUNK = 400
GATHER_W = 200
EMBED_SCALE = 8.0
NDIM = EMBED // LANES


def _body(idx_hbm, tok2_hbm, pos_hbm, out_hbm, idx_v, pidx_v, cb_v, rows2_v, out_v, pos_v, sem):
    c = lax.axis_index("c")
    s = lax.axis_index("s")
    wid = s * 2 + c
    n_rows = out_hbm.shape[0]
    rows_per_worker = n_rows // NUM_WORKERS
    chunks_per_worker = rows_per_worker // CHUNK
    batches_per_chunk = CHUNK // SEQ

    pltpu.sync_copy(pos_hbm, pos_v)

    iota16 = lax.iota(jnp.int32, LANES)

    def chunk_body(ci, _):
        base = wid * rows_per_worker + ci * CHUNK
        b0 = base // SEQ

        for jb in range(batches_per_chunk):
            pltpu.sync_copy(
                idx_hbm.at[pl.ds((b0 + jb) * SEQ_PAD, SEQ)],
                idx_v.at[pl.ds(jb * SEQ, SEQ)],
            )

        def pre_body(i, _):
            sl = pl.ds(i * LANES, LANES)
            v = idx_v[sl]
            pidx_v[sl] = lax.shift_right_logical(v, 1)
            cb_v[sl] = (v & 1) * EMBED
            return _

        lax.fori_loop(0, CHUNK // LANES, pre_body, None)

        copies = []
        for j in range(CHUNK // GATHER_W):
            copies.append(
                pltpu.async_copy(
                    tok2_hbm.at[pidx_v.at[pl.ds(j * GATHER_W, GATHER_W)]],
                    rows2_v.at[pl.ds(j * GATHER_W, GATHER_W)],
                    sem,
                )
            )
        for cp in copies:
            cp.wait()

        def pos_body(q, _):
            for p_off in range(2):
                p = 2 * q + p_off
                pv = [
                    pos_v[pl.ds(p * EMBED + d * LANES, LANES)]
                    for d in range(NDIM)
                ]
                for jb in range(batches_per_chunk):
                    r = jb * SEQ + p
                    cb = plsc.load_gather(
                        cb_v, [jnp.full((LANES,), r, jnp.int32)]
                    )
                    for d in range(NDIM):
                        cols = cb + (d * LANES + iota16)
                        v = plsc.load_gather(
                            rows2_v, [jnp.full((LANES,), r, jnp.int32), cols]
                        )
                        out_v[r, pl.ds(d * LANES, LANES)] = (
                            v * EMBED_SCALE + pv[d]
                        )
            return _

        lax.fori_loop(0, SEQ // 2, pos_body, None)

        pltpu.sync_copy(out_v, out_hbm.at[pl.ds(base, CHUNK)])
        return _

    lax.fori_loop(0, chunks_per_worker, chunk_body, None)


def kernel(inputs, token_table, position_table):
    batch, seq = inputs.shape
    vocab = token_table.shape[0]
    n_rows = batch * seq
    idx_pad = jnp.pad(inputs, ((0, 0), (0, SEQ_PAD - seq))).reshape(batch * SEQ_PAD)
    tok2 = token_table.reshape(vocab // 2, 2 * EMBED)
    pos1 = position_table.reshape(seq * EMBED)

    mesh = plsc.VectorSubcoreMesh(core_axis_name="c", subcore_axis_name="s")
    k = functools.partial(
        pl.kernel,
        mesh=mesh,
        out_type=jax.ShapeDtypeStruct((n_rows, EMBED), jnp.float32),
        scratch_types=[
            pltpu.VMEM((CHUNK,), jnp.int32),
            pltpu.VMEM((CHUNK,), jnp.int32),
            pltpu.VMEM((CHUNK,), jnp.int32),
            pltpu.VMEM((CHUNK, 2 * EMBED), jnp.float32),
            pltpu.VMEM((CHUNK, EMBED), jnp.float32),
            pltpu.VMEM((seq * EMBED,), jnp.float32),
            pltpu.SemaphoreType.DMA,
        ],
        compiler_params=pltpu.CompilerParams(
            use_tc_tiling_on_sc=True, needs_layout_passes=False
        ),
    )(_body)

    out = k(idx_pad, tok2, pos1)
    return out.reshape(batch, seq, EMBED)


# speedup vs baseline: 1.3786x; 1.3786x over previous
import functools

import jax
import jax.numpy as jnp
from jax import lax
from jax.experimental import pallas as pl
from jax.experimental.pallas import tpu as pltpu
from jax.experimental.pallas import tpu_sc as plsc

SEQ = 200
SEQ_PAD = 256
EMBED = 64
LANES = 16
NUM_WORKERS = 32
CHUNK = 400
GATHER_W = 200
EMBED_SCALE = 8.0
NDIM = EMBED // LANES


def _body(idx_hbm, tok2_hbm, pos_hbm, out_hbm, idx_v, pidx_v, cb_v, rows2_v, out_v, pos_v, sem):
    c = lax.axis_index("c")
    s = lax.axis_index("s")
    wid = s * 2 + c
    n_rows = out_hbm.shape[0]
    rows_per_worker = n_rows // NUM_WORKERS
    chunks_per_worker = rows_per_worker // CHUNK
    batches_per_chunk = CHUNK // SEQ

    pltpu.sync_copy(pos_hbm, pos_v)

    iota16 = lax.iota(jnp.int32, LANES)

    def chunk_body(ci, _):
        base = wid * rows_per_worker + ci * CHUNK
        b0 = base // SEQ

        for jb in range(batches_per_chunk):
            pltpu.sync_copy(
                idx_hbm.at[pl.ds((b0 + jb) * SEQ_PAD, SEQ)],
                idx_v.at[pl.ds(jb * SEQ, SEQ)],
            )

        def pre_body(i, _):
            sl = pl.ds(i * LANES, LANES)
            v = idx_v[sl]
            pidx_v[sl] = lax.shift_right_logical(v, 1)
            cb_v[sl] = (v & 1) * EMBED
            return _

        lax.fori_loop(0, CHUNK // LANES, pre_body, None)

        copies = []
        for j in range(CHUNK // GATHER_W):
            copies.append(
                pltpu.async_copy(
                    tok2_hbm.at[pidx_v.at[pl.ds(j * GATHER_W, GATHER_W)]],
                    rows2_v.at[pl.ds(j * GATHER_W, GATHER_W)],
                    sem,
                )
            )
        for cp in copies:
            cp.wait()

        def pos_body(q, _):
            for p_off in range(4):
                p = 4 * q + p_off
                pv = [
                    pos_v[pl.ds(p * EMBED + d * LANES, LANES)]
                    for d in range(NDIM)
                ]
                for jb in range(batches_per_chunk):
                    r = jb * SEQ + p
                    rv = plsc.load_gather(
                        cb_v, [jnp.full((LANES,), r, jnp.int32)]
                    )
                    odd = rv != 0
                    for d in range(NDIM):
                        lo = rows2_v[r, pl.ds(d * LANES, LANES)]
                        hi = rows2_v[r, pl.ds(EMBED + d * LANES, LANES)]
                        sel = jnp.where(odd, hi, lo)
                        out_v[r, pl.ds(d * LANES, LANES)] = (
                            sel * EMBED_SCALE + pv[d]
                        )
            return _

        lax.fori_loop(0, SEQ // 4, pos_body, None)

        pltpu.sync_copy(out_v, out_hbm.at[pl.ds(base, CHUNK)])
        return _

    lax.fori_loop(0, chunks_per_worker, chunk_body, None)


def kernel(inputs, token_table, position_table):
    batch, seq = inputs.shape
    vocab = token_table.shape[0]
    n_rows = batch * seq
    idx_pad = jnp.pad(inputs, ((0, 0), (0, SEQ_PAD - seq))).reshape(batch * SEQ_PAD)
    tok2 = token_table.reshape(vocab // 2, 2 * EMBED)
    pos1 = position_table.reshape(seq * EMBED)

    mesh = plsc.VectorSubcoreMesh(core_axis_name="c", subcore_axis_name="s")
    k = functools.partial(
        pl.kernel,
        mesh=mesh,
        out_type=jax.ShapeDtypeStruct((n_rows, EMBED), jnp.float32),
        scratch_types=[
            pltpu.VMEM((CHUNK,), jnp.int32),
            pltpu.VMEM((CHUNK,), jnp.int32),
            pltpu.VMEM((CHUNK,), jnp.int32),
            pltpu.VMEM((CHUNK, 2 * EMBED), jnp.float32),
            pltpu.VMEM((CHUNK, EMBED), jnp.float32),
            pltpu.VMEM((seq * EMBED,), jnp.float32),
            pltpu.SemaphoreType.DMA,
        ],
        compiler_params=pltpu.CompilerParams(
            use_tc_tiling_on_sc=True, needs_layout_passes=False
        ),
    )(_body)

    out = k(idx_pad, tok2, pos1)
    return out.reshape(batch, seq, EMBED)
